# Initial kernel scaffold; baseline (speedup 1.0000x reference)
import jax
import jax.numpy as jnp
from jax.experimental import pallas as pl


def kernel(x, edge_index, W1, b1, W2, b2, W3, b3):
    raise NotImplementedError("write your pallas kernel here")


# speedup vs baseline: 6.9505x; 6.9505x over previous
import functools

import jax
import jax.numpy as jnp
from jax import lax
from jax.experimental import pallas as pl
from jax.experimental.pallas import tpu as pltpu
from jax.experimental.pallas import tpu_sc as plsc

N = 10000
E = 320000
RP = 10240
DUMMY = N
NC = 2
NT = 16
CH = 128
EPT = E // (NC * NT)
KE = ((EPT + CH - 1) // CH + 7) // 8 * 8
RPT = RP // NT
BN = 1000

_MESH = dict(core_axis_name="c", subcore_axis_name="s", num_cores=NC,
             num_subcores=NT)


def _fill(ref, rows, value):
    width = ref.shape[1]
    vals = jnp.full((16,), value, jnp.float32)

    def body(i, _):
        for j in range(width // 16):
            ref[i, pl.ds(j * 16, 16)] = vals
        return 0

    lax.fori_loop(0, rows, body, 0)


def _deg_body(dst_hbm, out_hbm, acc):
    c = lax.axis_index("c")
    s = lax.axis_index("s")

    def scoped(dst_i, ones_v, zbuf):
        _fill(ones_v, CH, 1.0)
        _fill(zbuf, RPT // 8, 0.0)
        for kz in range(8):
            pltpu.sync_copy(zbuf, acc.at[pl.ds(s * RPT + kz * (RPT // 8),
                                               RPT // 8)])
        plsc.subcore_barrier()

        def chunk(j, _):
            pltpu.sync_copy(dst_hbm.at[c, s, j], dst_i)
            pltpu.sync_copy(ones_v, acc.at[dst_i], add=True)
            return 0

        lax.fori_loop(0, KE, chunk, 0)
        plsc.subcore_barrier()
        pltpu.sync_copy(acc.at[pl.ds(s * RPT, RPT)],
                        out_hbm.at[c, pl.ds(s * RPT, RPT)])

    pl.run_scoped(scoped,
                  pltpu.VMEM((CH,), jnp.int32),
                  pltpu.VMEM((CH, 128), jnp.float32),
                  pltpu.VMEM((RPT // 8, 128), jnp.float32))


@functools.cache
def _make_deg():
    return pl.kernel(
        _deg_body,
        out_type=jax.ShapeDtypeStruct((NC, RP, 128), jnp.float32),
        scratch_types=[
            pltpu.VMEM_SHARED((RP, 128), jnp.float32),
        ],
        mesh=plsc.VectorSubcoreMesh(**_MESH),
    )


def _prop_body(g_hbm, src_hbm, dst_hbm, out_hbm, acc, sem):
    c = lax.axis_index("c")
    s = lax.axis_index("s")

    def scoped(src_i, dst_i, rows_v, zbuf):
        _fill(zbuf, RPT // 8, 0.0)
        for kz in range(8):
            pltpu.sync_copy(zbuf, acc.at[pl.ds(s * RPT + kz * (RPT // 8),
                                               RPT // 8)])
        plsc.subcore_barrier()

        def chunk(j, _):
            pltpu.sync_copy(src_hbm.at[c, s, j], src_i)
            pltpu.sync_copy(dst_hbm.at[c, s, j], dst_i)
            pltpu.async_copy(g_hbm.at[src_i], rows_v, sem).wait()
            pltpu.sync_copy(rows_v, acc.at[dst_i], add=True)
            return 0

        lax.fori_loop(0, KE, chunk, 0)
        plsc.subcore_barrier()
        pltpu.sync_copy(acc.at[pl.ds(s * RPT, RPT)],
                        out_hbm.at[c, pl.ds(s * RPT, RPT)])

    pl.run_scoped(scoped,
                  pltpu.VMEM((CH,), jnp.int32),
                  pltpu.VMEM((CH,), jnp.int32),
                  pltpu.VMEM((CH, 128), jnp.float32),
                  pltpu.VMEM((RPT // 8, 128), jnp.float32))


@functools.cache
def _make_prop():
    return pl.kernel(
        _prop_body,
        out_type=jax.ShapeDtypeStruct((NC, RP, 128), jnp.float32),
        scratch_types=[
            pltpu.VMEM_SHARED((RP, 128), jnp.float32),
            pltpu.SemaphoreType.DMA,
        ],
        mesh=plsc.VectorSubcoreMesh(**_MESH),
    )


def _m1_body(x_ref, w_ref, dinv_ref, o_ref):
    g = jnp.dot(x_ref[...], w_ref[...], preferred_element_type=jnp.float32)
    o_ref[...] = g * dinv_ref[...]


_m1 = pl.pallas_call(
    _m1_body,
    grid=(N // BN,),
    in_specs=[
        pl.BlockSpec((BN, 128), lambda i: (i, 0)),
        pl.BlockSpec((128, 128), lambda i: (0, 0)),
        pl.BlockSpec((BN, 1), lambda i: (i, 0)),
    ],
    out_specs=pl.BlockSpec((BN, 128), lambda i: (i, 0)),
    out_shape=jax.ShapeDtypeStruct((N, 128), jnp.float32),
)


def _m2_body(s_ref, g_ref, dinv_ref, b_ref, w_ref, o_ref):
    h = (s_ref[0] + s_ref[1] + g_ref[...]) * dinv_ref[...] + b_ref[...]
    t = jnp.maximum(h, 0.0)
    g2 = jnp.dot(t, w_ref[...], preferred_element_type=jnp.float32)
    o_ref[...] = g2 * dinv_ref[...]


_m2 = pl.pallas_call(
    _m2_body,
    grid=(N // BN,),
    in_specs=[
        pl.BlockSpec((NC, BN, 128), lambda i: (0, i, 0)),
        pl.BlockSpec((BN, 128), lambda i: (i, 0)),
        pl.BlockSpec((BN, 1), lambda i: (i, 0)),
        pl.BlockSpec((1, 128), lambda i: (0, 0)),
        pl.BlockSpec((128, 128), lambda i: (0, 0)),
    ],
    out_specs=pl.BlockSpec((BN, 128), lambda i: (i, 0)),
    out_shape=jax.ShapeDtypeStruct((N, 128), jnp.float32),
)


def _m3_body(s_ref, g_ref, dinv_ref, b_ref, o_ref):
    h = (s_ref[0] + s_ref[1] + g_ref[...]) * dinv_ref[...] + b_ref[...]
    o_ref[...] = jnp.maximum(h, 0.0) * dinv_ref[...]


_m3 = pl.pallas_call(
    _m3_body,
    grid=(N // BN,),
    in_specs=[
        pl.BlockSpec((NC, BN, 128), lambda i: (0, i, 0)),
        pl.BlockSpec((BN, 128), lambda i: (i, 0)),
        pl.BlockSpec((BN, 1), lambda i: (i, 0)),
        pl.BlockSpec((1, 128), lambda i: (0, 0)),
    ],
    out_specs=pl.BlockSpec((BN, 128), lambda i: (i, 0)),
    out_shape=jax.ShapeDtypeStruct((N, 128), jnp.float32),
)


def _m4_body(s_ref, g_ref, dinv_ref, b_ref, w_ref, o_ref):
    h = (s_ref[0] + s_ref[1] + g_ref[...]) * dinv_ref[...]
    o_ref[...] = (jnp.dot(h, w_ref[...], preferred_element_type=jnp.float32)
                  + b_ref[...])


_m4 = pl.pallas_call(
    _m4_body,
    grid=(N // BN,),
    in_specs=[
        pl.BlockSpec((NC, BN, 128), lambda i: (0, i, 0)),
        pl.BlockSpec((BN, 128), lambda i: (i, 0)),
        pl.BlockSpec((BN, 1), lambda i: (i, 0)),
        pl.BlockSpec((1, 64), lambda i: (0, 0)),
        pl.BlockSpec((128, 64), lambda i: (0, 0)),
    ],
    out_specs=pl.BlockSpec((BN, 64), lambda i: (i, 0)),
    out_shape=jax.ShapeDtypeStruct((N, 64), jnp.float32),
)


def kernel(x, edge_index, W1, b1, W2, b2, W3, b3):
    src = edge_index[0].astype(jnp.int32)
    dst = edge_index[1].astype(jnp.int32)

    pade = KE * CH - EPT
    srce = jnp.concatenate(
        [src.reshape(NC, NT, EPT),
         jnp.zeros((NC, NT, pade), jnp.int32)], axis=2).reshape(NC, NT, KE, CH)
    dste = jnp.concatenate(
        [dst.reshape(NC, NT, EPT),
         jnp.full((NC, NT, pade), DUMMY, jnp.int32)], axis=2).reshape(
             NC, NT, KE, CH)

    prop = _make_prop()
    degp = _make_deg()(dste)
    deg = degp[0, :N, 0] + degp[1, :N, 0] + 1.0
    dinv = lax.rsqrt(deg).reshape(N, 1)

    g1 = _m1(x, W1, dinv)
    s1 = prop(g1, srce, dste)
    g2 = _m2(s1, g1, dinv, b1.reshape(1, 128), W2)
    s2 = prop(g2, srce, dste)
    g3 = _m3(s2, g2, dinv, b2.reshape(1, 128))
    s3 = prop(g3, srce, dste)
    return _m4(s3, g3, dinv, b3.reshape(1, 64), W3)


# speedup vs baseline: 8.6171x; 1.2398x over previous
import functools

import jax
import jax.numpy as jnp
from jax import lax
from jax.experimental import pallas as pl
from jax.experimental.pallas import tpu as pltpu
from jax.experimental.pallas import tpu_sc as plsc

N = 10000
E = 320000
RP = 10240
DUMMY = N
NC = 2
NT = 16
CH = 128
EPT = E // (NC * NT)
KE = ((EPT + CH - 1) // CH + 7) // 8 * 8
RPT = RP // NT
BN = 1000

_MESH = dict(core_axis_name="c", subcore_axis_name="s", num_cores=NC,
             num_subcores=NT)


def _fill(ref, rows, value):
    width = ref.shape[1]
    vals = jnp.full((16,), value, jnp.float32)

    def body(i, _):
        for j in range(width // 16):
            ref[i, pl.ds(j * 16, 16)] = vals
        return 0

    lax.fori_loop(0, rows, body, 0)


def _deg_body(dst_hbm, out_hbm, acc):
    c = lax.axis_index("c")
    s = lax.axis_index("s")

    def scoped(dst_i, ones_v, zbuf):
        _fill(ones_v, CH, 1.0)
        _fill(zbuf, RPT // 8, 0.0)
        for kz in range(8):
            pltpu.sync_copy(zbuf, acc.at[pl.ds(s * RPT + kz * (RPT // 8),
                                               RPT // 8)])
        plsc.subcore_barrier()

        def chunk(j, _):
            pltpu.sync_copy(dst_hbm.at[c, s, j], dst_i)
            pltpu.sync_copy(ones_v, acc.at[dst_i], add=True)
            return 0

        lax.fori_loop(0, KE, chunk, 0)
        plsc.subcore_barrier()
        pltpu.sync_copy(acc.at[pl.ds(s * RPT, RPT)],
                        out_hbm.at[c, pl.ds(s * RPT, RPT)])

    pl.run_scoped(scoped,
                  pltpu.VMEM((CH,), jnp.int32),
                  pltpu.VMEM((CH, 128), jnp.float32),
                  pltpu.VMEM((RPT // 8, 128), jnp.float32))


@functools.cache
def _make_deg():
    return pl.kernel(
        _deg_body,
        out_type=jax.ShapeDtypeStruct((NC, RP, 128), jnp.float32),
        scratch_types=[
            pltpu.VMEM_SHARED((RP, 128), jnp.float32),
        ],
        mesh=plsc.VectorSubcoreMesh(**_MESH),
    )


def _prop_body(g_hbm, src_hbm, dst_hbm, out_hbm, acc,
               sg0, sg1, ss0, ss1, sd0, sd1):
    c = lax.axis_index("c")
    s = lax.axis_index("s")

    def scoped(src0, src1, dst0, dst1, rows0, rows1, zbuf):
        _fill(zbuf, RPT // 8, 0.0)
        for kz in range(8):
            pltpu.sync_copy(zbuf, acc.at[pl.ds(s * RPT + kz * (RPT // 8),
                                               RPT // 8)])
        plsc.subcore_barrier()

        srcb = (src0, src1)
        dstb = (dst0, dst1)
        rowsb = (rows0, rows1)
        sg = (sg0, sg1)
        ssrc = (ss0, ss1)
        sdst = (sd0, sd1)

        def issue_idx(j, b):
            pltpu.async_copy(src_hbm.at[c, s, j], srcb[b], ssrc[b])
            pltpu.async_copy(dst_hbm.at[c, s, j], dstb[b], sdst[b])

        def wait_idx(j, b):
            pltpu.make_async_copy(src_hbm.at[c, s, j], srcb[b],
                                  ssrc[b]).wait()
            pltpu.make_async_copy(dst_hbm.at[c, s, j], dstb[b],
                                  sdst[b]).wait()

        def issue_gather(b):
            pltpu.async_copy(g_hbm.at[srcb[b]], rowsb[b], sg[b])

        def wait_gather(b):
            pltpu.make_async_copy(g_hbm.at[srcb[b]], rowsb[b], sg[b]).wait()

        issue_idx(0, 0)
        issue_idx(1, 1)
        wait_idx(0, 0)
        issue_gather(0)

        def pair(t, _):
            for bb in range(2):
                j = 2 * t + bb
                nb = 1 - bb

                @pl.when(j + 1 < KE)
                def _():
                    wait_idx(j + 1, nb)
                    issue_gather(nb)

                wait_gather(bb)
                pltpu.sync_copy(rowsb[bb], acc.at[dstb[bb]], add=True)

                @pl.when(j + 2 < KE)
                def _():
                    issue_idx(j + 2, bb)
            return 0

        lax.fori_loop(0, KE // 2, pair, 0)
        plsc.subcore_barrier()
        pltpu.sync_copy(acc.at[pl.ds(s * RPT, RPT)],
                        out_hbm.at[c, pl.ds(s * RPT, RPT)])

    pl.run_scoped(scoped,
                  pltpu.VMEM((CH,), jnp.int32),
                  pltpu.VMEM((CH,), jnp.int32),
                  pltpu.VMEM((CH,), jnp.int32),
                  pltpu.VMEM((CH,), jnp.int32),
                  pltpu.VMEM((CH, 128), jnp.float32),
                  pltpu.VMEM((CH, 128), jnp.float32),
                  pltpu.VMEM((RPT // 8, 128), jnp.float32))


@functools.cache
def _make_prop():
    return pl.kernel(
        _prop_body,
        out_type=jax.ShapeDtypeStruct((NC, RP, 128), jnp.float32),
        scratch_types=[
            pltpu.VMEM_SHARED((RP, 128), jnp.float32),
            pltpu.SemaphoreType.DMA,
            pltpu.SemaphoreType.DMA,
            pltpu.SemaphoreType.DMA,
            pltpu.SemaphoreType.DMA,
            pltpu.SemaphoreType.DMA,
            pltpu.SemaphoreType.DMA,
        ],
        mesh=plsc.VectorSubcoreMesh(**_MESH),
    )


def _m1_body(x_ref, w_ref, dinv_ref, o_ref):
    g = jnp.dot(x_ref[...], w_ref[...], preferred_element_type=jnp.float32)
    o_ref[...] = g * dinv_ref[...]


_m1 = pl.pallas_call(
    _m1_body,
    grid=(N // BN,),
    in_specs=[
        pl.BlockSpec((BN, 128), lambda i: (i, 0)),
        pl.BlockSpec((128, 128), lambda i: (0, 0)),
        pl.BlockSpec((BN, 1), lambda i: (i, 0)),
    ],
    out_specs=pl.BlockSpec((BN, 128), lambda i: (i, 0)),
    out_shape=jax.ShapeDtypeStruct((N, 128), jnp.float32),
)


def _m2_body(s_ref, g_ref, dinv_ref, b_ref, w_ref, o_ref):
    h = (s_ref[0] + s_ref[1] + g_ref[...]) * dinv_ref[...] + b_ref[...]
    t = jnp.maximum(h, 0.0)
    g2 = jnp.dot(t, w_ref[...], preferred_element_type=jnp.float32)
    o_ref[...] = g2 * dinv_ref[...]


_m2 = pl.pallas_call(
    _m2_body,
    grid=(N // BN,),
    in_specs=[
        pl.BlockSpec((NC, BN, 128), lambda i: (0, i, 0)),
        pl.BlockSpec((BN, 128), lambda i: (i, 0)),
        pl.BlockSpec((BN, 1), lambda i: (i, 0)),
        pl.BlockSpec((1, 128), lambda i: (0, 0)),
        pl.BlockSpec((128, 128), lambda i: (0, 0)),
    ],
    out_specs=pl.BlockSpec((BN, 128), lambda i: (i, 0)),
    out_shape=jax.ShapeDtypeStruct((N, 128), jnp.float32),
)


def _m3_body(s_ref, g_ref, dinv_ref, b_ref, o_ref):
    h = (s_ref[0] + s_ref[1] + g_ref[...]) * dinv_ref[...] + b_ref[...]
    o_ref[...] = jnp.maximum(h, 0.0) * dinv_ref[...]


_m3 = pl.pallas_call(
    _m3_body,
    grid=(N // BN,),
    in_specs=[
        pl.BlockSpec((NC, BN, 128), lambda i: (0, i, 0)),
        pl.BlockSpec((BN, 128), lambda i: (i, 0)),
        pl.BlockSpec((BN, 1), lambda i: (i, 0)),
        pl.BlockSpec((1, 128), lambda i: (0, 0)),
    ],
    out_specs=pl.BlockSpec((BN, 128), lambda i: (i, 0)),
    out_shape=jax.ShapeDtypeStruct((N, 128), jnp.float32),
)


def _m4_body(s_ref, g_ref, dinv_ref, b_ref, w_ref, o_ref):
    h = (s_ref[0] + s_ref[1] + g_ref[...]) * dinv_ref[...]
    o_ref[...] = (jnp.dot(h, w_ref[...], preferred_element_type=jnp.float32)
                  + b_ref[...])


_m4 = pl.pallas_call(
    _m4_body,
    grid=(N // BN,),
    in_specs=[
        pl.BlockSpec((NC, BN, 128), lambda i: (0, i, 0)),
        pl.BlockSpec((BN, 128), lambda i: (i, 0)),
        pl.BlockSpec((BN, 1), lambda i: (i, 0)),
        pl.BlockSpec((1, 64), lambda i: (0, 0)),
        pl.BlockSpec((128, 64), lambda i: (0, 0)),
    ],
    out_specs=pl.BlockSpec((BN, 64), lambda i: (i, 0)),
    out_shape=jax.ShapeDtypeStruct((N, 64), jnp.float32),
)


def kernel(x, edge_index, W1, b1, W2, b2, W3, b3):
    src = edge_index[0].astype(jnp.int32)
    dst = edge_index[1].astype(jnp.int32)

    pade = KE * CH - EPT
    srce = jnp.concatenate(
        [src.reshape(NC, NT, EPT),
         jnp.zeros((NC, NT, pade), jnp.int32)], axis=2).reshape(NC, NT, KE, CH)
    dste = jnp.concatenate(
        [dst.reshape(NC, NT, EPT),
         jnp.full((NC, NT, pade), DUMMY, jnp.int32)], axis=2).reshape(
             NC, NT, KE, CH)

    prop = _make_prop()
    degp = _make_deg()(dste)
    deg = degp[0, :N, 0] + degp[1, :N, 0] + 1.0
    dinv = lax.rsqrt(deg).reshape(N, 1)

    g1 = _m1(x, W1, dinv)
    s1 = prop(g1, srce, dste)
    g2 = _m2(s1, g1, dinv, b1.reshape(1, 128), W2)
    s2 = prop(g2, srce, dste)
    g3 = _m3(s2, g2, dinv, b2.reshape(1, 128))
    s3 = prop(g3, srce, dste)
    return _m4(s3, g3, dinv, b3.reshape(1, 64), W3)


# speedup vs baseline: 8.6210x; 1.0005x over previous
import functools

import jax
import jax.numpy as jnp
from jax import lax
from jax.experimental import pallas as pl
from jax.experimental.pallas import tpu as pltpu
from jax.experimental.pallas import tpu_sc as plsc

N = 10000
E = 320000
RP = 10240
DUMMY = N
NC = 2
NT = 16
CH = 128
EPT = E // (NC * NT)
KE = ((EPT + CH - 1) // CH + 7) // 8 * 8
RPT = RP // NT
BN = 1000

_MESH = dict(core_axis_name="c", subcore_axis_name="s", num_cores=NC,
             num_subcores=NT)


def _fill(ref, rows, value):
    width = ref.shape[1]
    vals = jnp.full((16,), value, jnp.float32)

    def body(i, _):
        for j in range(width // 16):
            ref[i, pl.ds(j * 16, 16)] = vals
        return 0

    lax.fori_loop(0, rows, body, 0)


def _deg_body(dst_hbm, out_hbm, acc):
    c = lax.axis_index("c")
    s = lax.axis_index("s")

    def scoped(dst_i, ones_v, zbuf):
        _fill(ones_v, CH, 1.0)
        _fill(zbuf, RPT // 8, 0.0)
        for kz in range(8):
            pltpu.sync_copy(zbuf, acc.at[pl.ds(s * RPT + kz * (RPT // 8),
                                               RPT // 8)])
        plsc.subcore_barrier()

        def chunk(j, _):
            pltpu.sync_copy(dst_hbm.at[c, s, j], dst_i)
            pltpu.sync_copy(ones_v, acc.at[dst_i], add=True)
            return 0

        lax.fori_loop(0, KE, chunk, 0)
        plsc.subcore_barrier()
        pltpu.sync_copy(acc.at[pl.ds(s * RPT, RPT)],
                        out_hbm.at[c, pl.ds(s * RPT, RPT)])

    pl.run_scoped(scoped,
                  pltpu.VMEM((CH,), jnp.int32),
                  pltpu.VMEM((CH, 128), jnp.float32),
                  pltpu.VMEM((RPT // 8, 128), jnp.float32))


@functools.cache
def _make_deg():
    return pl.kernel(
        _deg_body,
        out_type=jax.ShapeDtypeStruct((NC, RP, 128), jnp.float32),
        scratch_types=[
            pltpu.VMEM_SHARED((RP, 128), jnp.float32),
        ],
        mesh=plsc.VectorSubcoreMesh(**_MESH),
    )


def _prop_body(g_hbm, src_hbm, dst_hbm, out_hbm, acc,
               sg0, sg1, ss0, ss1, sd0, sd1):
    c = lax.axis_index("c")
    s = lax.axis_index("s")

    def scoped(src0, src1, dst0, dst1, rows0, rows1, zbuf):
        _fill(zbuf, RPT // 8, 0.0)
        for kz in range(8):
            pltpu.sync_copy(zbuf, acc.at[pl.ds(s * RPT + kz * (RPT // 8),
                                               RPT // 8)])
        plsc.subcore_barrier()

        srcb = (src0, src1)
        dstb = (dst0, dst1)
        rowsb = (rows0, rows1)
        sg = (sg0, sg1)
        ssrc = (ss0, ss1)
        sdst = (sd0, sd1)

        def issue_idx(j, b):
            pltpu.async_copy(src_hbm.at[c, s, j], srcb[b], ssrc[b])
            pltpu.async_copy(dst_hbm.at[c, s, j], dstb[b], sdst[b])

        def wait_idx(j, b):
            pltpu.make_async_copy(src_hbm.at[c, s, j], srcb[b],
                                  ssrc[b]).wait()
            pltpu.make_async_copy(dst_hbm.at[c, s, j], dstb[b],
                                  sdst[b]).wait()

        def issue_gather(b):
            pltpu.async_copy(g_hbm.at[srcb[b]], rowsb[b], sg[b])

        def wait_gather(b):
            pltpu.make_async_copy(g_hbm.at[srcb[b]], rowsb[b], sg[b]).wait()

        issue_idx(0, 0)
        issue_idx(1, 1)
        wait_idx(0, 0)
        issue_gather(0)

        def pair(t, _):
            for bb in range(2):
                j = 2 * t + bb
                nb = 1 - bb

                @pl.when(j + 1 < KE)
                def _():
                    wait_idx(j + 1, nb)
                    issue_gather(nb)

                wait_gather(bb)
                pltpu.sync_copy(rowsb[bb], acc.at[dstb[bb]], add=True)

                @pl.when(j + 2 < KE)
                def _():
                    issue_idx(j + 2, bb)
            return 0

        lax.fori_loop(0, KE // 2, pair, 0)
        plsc.subcore_barrier()
        pltpu.sync_copy(acc.at[pl.ds(s * RPT, RPT)],
                        out_hbm.at[c, pl.ds(s * RPT, RPT)])

    pl.run_scoped(scoped,
                  pltpu.VMEM((CH,), jnp.int32),
                  pltpu.VMEM((CH,), jnp.int32),
                  pltpu.VMEM((CH,), jnp.int32),
                  pltpu.VMEM((CH,), jnp.int32),
                  pltpu.VMEM((CH, 128), jnp.float32),
                  pltpu.VMEM((CH, 128), jnp.float32),
                  pltpu.VMEM((RPT // 8, 128), jnp.float32))


@functools.cache
def _make_prop():
    return pl.kernel(
        _prop_body,
        out_type=jax.ShapeDtypeStruct((NC, RP, 128), jnp.float32),
        scratch_types=[pltpu.VMEM_SHARED((RP, 128), jnp.float32)]
        + [pltpu.SemaphoreType.DMA] * 6,
        mesh=plsc.VectorSubcoreMesh(**_MESH),
    )


def _m1_body(x_ref, w_ref, dinv_ref, o_ref):
    g = jnp.dot(x_ref[...], w_ref[...], preferred_element_type=jnp.float32)
    o_ref[...] = g * dinv_ref[...]


_m1 = pl.pallas_call(
    _m1_body,
    grid=(N // BN,),
    in_specs=[
        pl.BlockSpec((BN, 128), lambda i: (i, 0)),
        pl.BlockSpec((128, 128), lambda i: (0, 0)),
        pl.BlockSpec((BN, 1), lambda i: (i, 0)),
    ],
    out_specs=pl.BlockSpec((BN, 128), lambda i: (i, 0)),
    out_shape=jax.ShapeDtypeStruct((N, 128), jnp.float32),
)


def _m2_body(s_ref, g_ref, dinv_ref, b_ref, w_ref, o_ref):
    h = (s_ref[0] + s_ref[1] + g_ref[...]) * dinv_ref[...] + b_ref[...]
    t = jnp.maximum(h, 0.0)
    g2 = jnp.dot(t, w_ref[...], preferred_element_type=jnp.float32)
    o_ref[...] = g2 * dinv_ref[...]


_m2 = pl.pallas_call(
    _m2_body,
    grid=(N // BN,),
    in_specs=[
        pl.BlockSpec((NC, BN, 128), lambda i: (0, i, 0)),
        pl.BlockSpec((BN, 128), lambda i: (i, 0)),
        pl.BlockSpec((BN, 1), lambda i: (i, 0)),
        pl.BlockSpec((1, 128), lambda i: (0, 0)),
        pl.BlockSpec((128, 128), lambda i: (0, 0)),
    ],
    out_specs=pl.BlockSpec((BN, 128), lambda i: (i, 0)),
    out_shape=jax.ShapeDtypeStruct((N, 128), jnp.float32),
)


def _m3_body(s_ref, g_ref, dinv_ref, b_ref, o_ref):
    h = (s_ref[0] + s_ref[1] + g_ref[...]) * dinv_ref[...] + b_ref[...]
    o_ref[...] = jnp.maximum(h, 0.0) * dinv_ref[...]


_m3 = pl.pallas_call(
    _m3_body,
    grid=(N // BN,),
    in_specs=[
        pl.BlockSpec((NC, BN, 128), lambda i: (0, i, 0)),
        pl.BlockSpec((BN, 128), lambda i: (i, 0)),
        pl.BlockSpec((BN, 1), lambda i: (i, 0)),
        pl.BlockSpec((1, 128), lambda i: (0, 0)),
    ],
    out_specs=pl.BlockSpec((BN, 128), lambda i: (i, 0)),
    out_shape=jax.ShapeDtypeStruct((N, 128), jnp.float32),
)


def _m4_body(s_ref, g_ref, dinv_ref, b_ref, w_ref, o_ref):
    h = (s_ref[0] + s_ref[1] + g_ref[...]) * dinv_ref[...]
    o_ref[...] = (jnp.dot(h, w_ref[...], preferred_element_type=jnp.float32)
                  + b_ref[...])


_m4 = pl.pallas_call(
    _m4_body,
    grid=(N // BN,),
    in_specs=[
        pl.BlockSpec((NC, BN, 128), lambda i: (0, i, 0)),
        pl.BlockSpec((BN, 128), lambda i: (i, 0)),
        pl.BlockSpec((BN, 1), lambda i: (i, 0)),
        pl.BlockSpec((1, 64), lambda i: (0, 0)),
        pl.BlockSpec((128, 64), lambda i: (0, 0)),
    ],
    out_specs=pl.BlockSpec((BN, 64), lambda i: (i, 0)),
    out_shape=jax.ShapeDtypeStruct((N, 64), jnp.float32),
)


def kernel(x, edge_index, W1, b1, W2, b2, W3, b3):
    src = edge_index[0].astype(jnp.int32)
    dst = edge_index[1].astype(jnp.int32)

    pade = KE * CH - EPT
    srce = jnp.concatenate(
        [src.reshape(NC, NT, EPT),
         jnp.zeros((NC, NT, pade), jnp.int32)], axis=2).reshape(NC, NT, KE, CH)
    dste = jnp.concatenate(
        [dst.reshape(NC, NT, EPT),
         jnp.full((NC, NT, pade), DUMMY, jnp.int32)], axis=2).reshape(
             NC, NT, KE, CH)

    prop = _make_prop()
    degp = _make_deg()(dste)
    deg = degp[0, :N, 0] + degp[1, :N, 0] + 1.0
    dinv = lax.rsqrt(deg).reshape(N, 1)

    g1 = _m1(x, W1, dinv)
    s1 = prop(g1, srce, dste)
    g2 = _m2(s1, g1, dinv, b1.reshape(1, 128), W2)
    s2 = prop(g2, srce, dste)
    g3 = _m3(s2, g2, dinv, b2.reshape(1, 128))
    s3 = prop(g3, srce, dste)
    return _m4(s3, g3, dinv, b3.reshape(1, 64), W3)


# speedup vs baseline: 8.8268x; 1.0239x over previous
import functools

import jax
import jax.numpy as jnp
from jax import lax
from jax.experimental import pallas as pl
from jax.experimental.pallas import tpu as pltpu
from jax.experimental.pallas import tpu_sc as plsc

N = 10000
E = 320000
RP = 10240
DUMMY = N
NC = 2
NT = 16
CH = 128
EPT = E // (NC * NT)
KE = ((EPT + CH - 1) // CH + 7) // 8 * 8
RPT = RP // NT
BN = 1000

_MESH = dict(core_axis_name="c", subcore_axis_name="s", num_cores=NC,
             num_subcores=NT)


def _fill(ref, rows, value):
    width = ref.shape[1]
    vals = jnp.full((16,), value, jnp.float32)

    def body(i, _):
        for j in range(width // 16):
            ref[i, pl.ds(j * 16, 16)] = vals
        return 0

    lax.fori_loop(0, rows, body, 0)


def _deg_body(dst_hbm, out_hbm, acc, dc0, dc1, dc2, dc3, di0, di1, di2, di3):
    c = lax.axis_index("c")
    s = lax.axis_index("s")

    def scoped(dst0, dst1, dst2, dst3, ones_v, zbuf):
        dstb = (dst0, dst1, dst2, dst3)
        sc = (dc0, dc1, dc2, dc3)
        si = (di0, di1, di2, di3)
        _fill(ones_v, CH, 1.0)
        _fill(zbuf, RPT // 8, 0.0)
        for kz in range(8):
            pltpu.sync_copy(zbuf, acc.at[pl.ds(s * RPT + kz * (RPT // 8),
                                               RPT // 8)])
        plsc.subcore_barrier()

        def issue_idx(j, b):
            pltpu.async_copy(dst_hbm.at[c, s, j], dstb[b], si[b])

        def wait_idx(j, b):
            pltpu.make_async_copy(dst_hbm.at[c, s, j], dstb[b], si[b]).wait()

        def issue_scatter(b):
            pltpu.async_copy(ones_v, acc.at[dstb[b]], sc[b], add=True)

        def wait_scatter(b):
            pltpu.make_async_copy(ones_v, acc.at[dstb[b]], sc[b]).wait()

        issue_idx(0, 0)
        issue_idx(1, 1)

        def quad(t, _):
            for bb in range(4):
                j = 4 * t + bb

                @pl.when(j >= 2)
                def _():
                    wait_scatter((bb + 2) % 4)

                wait_idx(j, bb)
                issue_scatter(bb)

                @pl.when(j + 2 < KE)
                def _():
                    issue_idx(j + 2, (bb + 2) % 4)
            return 0

        lax.fori_loop(0, KE // 4, quad, 0)
        for j in range(KE - 2, KE):
            wait_scatter(j % 4)
        plsc.subcore_barrier()
        pltpu.sync_copy(acc.at[pl.ds(s * RPT, RPT)],
                        out_hbm.at[c, pl.ds(s * RPT, RPT)])

    pl.run_scoped(scoped,
                  pltpu.VMEM((CH,), jnp.int32),
                  pltpu.VMEM((CH,), jnp.int32),
                  pltpu.VMEM((CH,), jnp.int32),
                  pltpu.VMEM((CH,), jnp.int32),
                  pltpu.VMEM((CH, 128), jnp.float32),
                  pltpu.VMEM((RPT // 8, 128), jnp.float32))


@functools.cache
def _make_deg():
    return pl.kernel(
        _deg_body,
        out_type=jax.ShapeDtypeStruct((NC, RP, 128), jnp.float32),
        scratch_types=[pltpu.VMEM_SHARED((RP, 128), jnp.float32)]
        + [pltpu.SemaphoreType.DMA] * 8,
        mesh=plsc.VectorSubcoreMesh(**_MESH),
    )


def _prop_body(g_hbm, src_hbm, dst_hbm, out_hbm, acc,
               sg0, sg1, ss0, ss1, sd0, sd1):
    c = lax.axis_index("c")
    s = lax.axis_index("s")

    def scoped(src0, src1, dst0, dst1, rows0, rows1, zbuf):
        _fill(zbuf, RPT // 8, 0.0)
        for kz in range(8):
            pltpu.sync_copy(zbuf, acc.at[pl.ds(s * RPT + kz * (RPT // 8),
                                               RPT // 8)])
        plsc.subcore_barrier()

        srcb = (src0, src1)
        dstb = (dst0, dst1)
        rowsb = (rows0, rows1)
        sg = (sg0, sg1)
        ssrc = (ss0, ss1)
        sdst = (sd0, sd1)

        def issue_idx(j, b):
            pltpu.async_copy(src_hbm.at[c, s, j], srcb[b], ssrc[b])
            pltpu.async_copy(dst_hbm.at[c, s, j], dstb[b], sdst[b])

        def wait_idx(j, b):
            pltpu.make_async_copy(src_hbm.at[c, s, j], srcb[b],
                                  ssrc[b]).wait()
            pltpu.make_async_copy(dst_hbm.at[c, s, j], dstb[b],
                                  sdst[b]).wait()

        def issue_gather(b):
            pltpu.async_copy(g_hbm.at[srcb[b]], rowsb[b], sg[b])

        def wait_gather(b):
            pltpu.make_async_copy(g_hbm.at[srcb[b]], rowsb[b], sg[b]).wait()

        issue_idx(0, 0)
        issue_idx(1, 1)
        wait_idx(0, 0)
        issue_gather(0)

        def pair(t, _):
            for bb in range(2):
                j = 2 * t + bb
                nb = 1 - bb

                @pl.when(j + 1 < KE)
                def _():
                    wait_idx(j + 1, nb)
                    issue_gather(nb)

                wait_gather(bb)
                pltpu.sync_copy(rowsb[bb], acc.at[dstb[bb]], add=True)

                @pl.when(j + 2 < KE)
                def _():
                    issue_idx(j + 2, bb)
            return 0

        lax.fori_loop(0, KE // 2, pair, 0)
        plsc.subcore_barrier()
        pltpu.sync_copy(acc.at[pl.ds(s * RPT, RPT)],
                        out_hbm.at[c, pl.ds(s * RPT, RPT)])

    pl.run_scoped(scoped,
                  pltpu.VMEM((CH,), jnp.int32),
                  pltpu.VMEM((CH,), jnp.int32),
                  pltpu.VMEM((CH,), jnp.int32),
                  pltpu.VMEM((CH,), jnp.int32),
                  pltpu.VMEM((CH, 128), jnp.float32),
                  pltpu.VMEM((CH, 128), jnp.float32),
                  pltpu.VMEM((RPT // 8, 128), jnp.float32))


@functools.cache
def _make_prop():
    return pl.kernel(
        _prop_body,
        out_type=jax.ShapeDtypeStruct((NC, RP, 128), jnp.float32),
        scratch_types=[pltpu.VMEM_SHARED((RP, 128), jnp.float32)]
        + [pltpu.SemaphoreType.DMA] * 6,
        mesh=plsc.VectorSubcoreMesh(**_MESH),
    )


def _m1_body(x_ref, w_ref, dinv_ref, o_ref):
    g = jnp.dot(x_ref[...], w_ref[...], preferred_element_type=jnp.float32)
    o_ref[...] = g * dinv_ref[...]


_m1 = pl.pallas_call(
    _m1_body,
    grid=(N // BN,),
    in_specs=[
        pl.BlockSpec((BN, 128), lambda i: (i, 0)),
        pl.BlockSpec((128, 128), lambda i: (0, 0)),
        pl.BlockSpec((BN, 1), lambda i: (i, 0)),
    ],
    out_specs=pl.BlockSpec((BN, 128), lambda i: (i, 0)),
    out_shape=jax.ShapeDtypeStruct((N, 128), jnp.float32),
)


def _m2_body(s_ref, g_ref, dinv_ref, b_ref, w_ref, o_ref):
    h = (s_ref[0] + s_ref[1] + g_ref[...]) * dinv_ref[...] + b_ref[...]
    t = jnp.maximum(h, 0.0)
    g2 = jnp.dot(t, w_ref[...], preferred_element_type=jnp.float32)
    o_ref[...] = g2 * dinv_ref[...]


_m2 = pl.pallas_call(
    _m2_body,
    grid=(N // BN,),
    in_specs=[
        pl.BlockSpec((NC, BN, 128), lambda i: (0, i, 0)),
        pl.BlockSpec((BN, 128), lambda i: (i, 0)),
        pl.BlockSpec((BN, 1), lambda i: (i, 0)),
        pl.BlockSpec((1, 128), lambda i: (0, 0)),
        pl.BlockSpec((128, 128), lambda i: (0, 0)),
    ],
    out_specs=pl.BlockSpec((BN, 128), lambda i: (i, 0)),
    out_shape=jax.ShapeDtypeStruct((N, 128), jnp.float32),
)


def _m3_body(s_ref, g_ref, dinv_ref, b_ref, o_ref):
    h = (s_ref[0] + s_ref[1] + g_ref[...]) * dinv_ref[...] + b_ref[...]
    o_ref[...] = jnp.maximum(h, 0.0) * dinv_ref[...]


_m3 = pl.pallas_call(
    _m3_body,
    grid=(N // BN,),
    in_specs=[
        pl.BlockSpec((NC, BN, 128), lambda i: (0, i, 0)),
        pl.BlockSpec((BN, 128), lambda i: (i, 0)),
        pl.BlockSpec((BN, 1), lambda i: (i, 0)),
        pl.BlockSpec((1, 128), lambda i: (0, 0)),
    ],
    out_specs=pl.BlockSpec((BN, 128), lambda i: (i, 0)),
    out_shape=jax.ShapeDtypeStruct((N, 128), jnp.float32),
)


def _m4_body(s_ref, g_ref, dinv_ref, b_ref, w_ref, o_ref):
    h = (s_ref[0] + s_ref[1] + g_ref[...]) * dinv_ref[...]
    o_ref[...] = (jnp.dot(h, w_ref[...], preferred_element_type=jnp.float32)
                  + b_ref[...])


_m4 = pl.pallas_call(
    _m4_body,
    grid=(N // BN,),
    in_specs=[
        pl.BlockSpec((NC, BN, 128), lambda i: (0, i, 0)),
        pl.BlockSpec((BN, 128), lambda i: (i, 0)),
        pl.BlockSpec((BN, 1), lambda i: (i, 0)),
        pl.BlockSpec((1, 64), lambda i: (0, 0)),
        pl.BlockSpec((128, 64), lambda i: (0, 0)),
    ],
    out_specs=pl.BlockSpec((BN, 64), lambda i: (i, 0)),
    out_shape=jax.ShapeDtypeStruct((N, 64), jnp.float32),
)


def kernel(x, edge_index, W1, b1, W2, b2, W3, b3):
    src = edge_index[0].astype(jnp.int32)
    dst = edge_index[1].astype(jnp.int32)

    pade = KE * CH - EPT
    srce = jnp.concatenate(
        [src.reshape(NC, NT, EPT),
         jnp.zeros((NC, NT, pade), jnp.int32)], axis=2).reshape(NC, NT, KE, CH)
    dste = jnp.concatenate(
        [dst.reshape(NC, NT, EPT),
         jnp.full((NC, NT, pade), DUMMY, jnp.int32)], axis=2).reshape(
             NC, NT, KE, CH)

    prop = _make_prop()
    degp = _make_deg()(dste)
    deg = degp[0, :N, 0] + degp[1, :N, 0] + 1.0
    dinv = lax.rsqrt(deg).reshape(N, 1)

    g1 = _m1(x, W1, dinv)
    s1 = prop(g1, srce, dste)
    g2 = _m2(s1, g1, dinv, b1.reshape(1, 128), W2)
    s2 = prop(g2, srce, dste)
    g3 = _m3(s2, g2, dinv, b2.reshape(1, 128))
    s3 = prop(g3, srce, dste)
    return _m4(s3, g3, dinv, b3.reshape(1, 64), W3)
